# baseline (speedup 1.0000x reference)
import jax
import jax.numpy as jnp
from jax import lax
from jax.experimental import pallas as pl
from jax.experimental.pallas import tpu as pltpu

_NUM_EXPERTS = 16
_CAPACITY = 128.0
_CHUNK = 2048


def _router_body(h_ref, wt_ref, tri_ref, exp_ref, pm_ref, logit_ref,
                 carry_ref):
    c = pl.program_id(1)

    @pl.when(c == 0)
    def _():
        carry_ref[...] = jnp.zeros_like(carry_ref)

    h = h_ref[0]
    wt = wt_ref[...]
    logits = jnp.dot(h, wt, preferred_element_type=jnp.float32)
    logit_ref[0] = logits

    m = jnp.max(logits, axis=-1, keepdims=True)
    s = jnp.sum(jnp.exp(logits - m), axis=-1, keepdims=True)
    pm_ref[0] = 1.0 / s

    eq = (logits >= m).astype(jnp.bfloat16)
    rr = lax.broadcasted_iota(jnp.int32, (_NUM_EXPERTS, _NUM_EXPERTS), 0)
    cc = lax.broadcasted_iota(jnp.int32, (_NUM_EXPERTS, _NUM_EXPERTS), 1)
    upper = (rr <= cc).astype(jnp.bfloat16)
    pfx = jnp.dot(eq, upper, preferred_element_type=jnp.float32)
    ohb = eq * (pfx <= 1.0).astype(jnp.bfloat16)
    ohf = ohb.astype(jnp.float32)

    pri = (jnp.dot(tri_ref[...], ohb, preferred_element_type=jnp.float32)
           + carry_ref[...])

    keep = (pri <= _CAPACITY).astype(jnp.float32)
    exp_ref[0] = (ohf * keep).astype(jnp.int32)
    carry_ref[...] = carry_ref[...] + jnp.sum(ohf, axis=0, keepdims=True)


def kernel(hidden_states, W):
    B, S, H = hidden_states.shape
    wt = W.T
    tri = jnp.tril(jnp.ones((_CHUNK, _CHUNK), jnp.bfloat16))
    grid = (B, S // _CHUNK)
    out_shape = (
        jax.ShapeDtypeStruct((B, S, _NUM_EXPERTS), jnp.int32),
        jax.ShapeDtypeStruct((B, S, 1), jnp.float32),
        jax.ShapeDtypeStruct((B, S, _NUM_EXPERTS), jnp.float32),
    )
    return pl.pallas_call(
        _router_body,
        grid=grid,
        in_specs=[
            pl.BlockSpec((1, _CHUNK, H), lambda b, c: (b, c, 0)),
            pl.BlockSpec((H, _NUM_EXPERTS), lambda b, c: (0, 0)),
            pl.BlockSpec((_CHUNK, _CHUNK), lambda b, c: (0, 0)),
        ],
        out_specs=(
            pl.BlockSpec((1, _CHUNK, _NUM_EXPERTS), lambda b, c: (b, c, 0)),
            pl.BlockSpec((1, _CHUNK, 1), lambda b, c: (b, c, 0)),
            pl.BlockSpec((1, _CHUNK, _NUM_EXPERTS), lambda b, c: (b, c, 0)),
        ),
        out_shape=out_shape,
        scratch_shapes=[pltpu.VMEM((1, _NUM_EXPERTS), jnp.float32)],
        compiler_params=pltpu.CompilerParams(
            dimension_semantics=("arbitrary", "arbitrary")),
    )(hidden_states, wt, tri)


# speedup vs baseline: 1.4123x; 1.4123x over previous
import jax
import jax.numpy as jnp
from jax import lax
from jax.experimental import pallas as pl
from jax.experimental.pallas import tpu as pltpu

_NUM_EXPERTS = 16
_CAPACITY = 128.0
_CHUNK = 512


def _cumsum_rows(x):
    k = 1
    while k < _CHUNK:
        x = x + jnp.pad(x[:-k], ((k, 0), (0, 0)))
        k *= 2
    return x


def _router_body(h_ref, wt_ref, exp_ref, pm_ref, logit_ref, carry_ref):
    c = pl.program_id(1)

    @pl.when(c == 0)
    def _():
        carry_ref[...] = jnp.zeros_like(carry_ref)

    h = h_ref[0]
    wt = wt_ref[...]
    logits = jnp.dot(h, wt, preferred_element_type=jnp.float32)
    logit_ref[0] = logits

    m = jnp.max(logits, axis=-1, keepdims=True)
    s = jnp.sum(jnp.exp(logits - m), axis=-1, keepdims=True)
    pm_ref[0] = 1.0 / s

    eq = (logits >= m).astype(jnp.bfloat16)
    rr = lax.broadcasted_iota(jnp.int32, (_NUM_EXPERTS, _NUM_EXPERTS), 0)
    cc = lax.broadcasted_iota(jnp.int32, (_NUM_EXPERTS, _NUM_EXPERTS), 1)
    upper = (rr <= cc).astype(jnp.bfloat16)
    pfx = jnp.dot(eq, upper, preferred_element_type=jnp.float32)
    ohf = (eq * (pfx <= 1.0).astype(jnp.bfloat16)).astype(jnp.float32)

    pri = _cumsum_rows(ohf) + carry_ref[...]

    keep = (pri <= _CAPACITY).astype(jnp.float32)
    exp_ref[0] = (ohf * keep).astype(jnp.int32)
    carry_ref[...] = carry_ref[...] + jnp.sum(ohf, axis=0, keepdims=True)


def kernel(hidden_states, W):
    B, S, H = hidden_states.shape
    wt = W.T
    grid = (B, S // _CHUNK)
    out_shape = (
        jax.ShapeDtypeStruct((B, S, _NUM_EXPERTS), jnp.int32),
        jax.ShapeDtypeStruct((B, S, 1), jnp.float32),
        jax.ShapeDtypeStruct((B, S, _NUM_EXPERTS), jnp.float32),
    )
    return pl.pallas_call(
        _router_body,
        grid=grid,
        in_specs=[
            pl.BlockSpec((1, _CHUNK, H), lambda b, c: (b, c, 0)),
            pl.BlockSpec((H, _NUM_EXPERTS), lambda b, c: (0, 0)),
        ],
        out_specs=(
            pl.BlockSpec((1, _CHUNK, _NUM_EXPERTS), lambda b, c: (b, c, 0)),
            pl.BlockSpec((1, _CHUNK, 1), lambda b, c: (b, c, 0)),
            pl.BlockSpec((1, _CHUNK, _NUM_EXPERTS), lambda b, c: (b, c, 0)),
        ),
        out_shape=out_shape,
        scratch_shapes=[pltpu.VMEM((1, _NUM_EXPERTS), jnp.float32)],
        compiler_params=pltpu.CompilerParams(
            dimension_semantics=("arbitrary", "arbitrary")),
    )(hidden_states, wt)


# speedup vs baseline: 1.5309x; 1.0840x over previous
import jax
import jax.numpy as jnp
from jax import lax
from jax.experimental import pallas as pl
from jax.experimental.pallas import tpu as pltpu

_NUM_EXPERTS = 16
_CAPACITY = 128.0
_CHUNK = 1024


def _cumsum_rows(x):
    k = 1
    while k < _CHUNK:
        x = x + jnp.pad(x[:-k], ((k, 0), (0, 0)))
        k *= 2
    return x


def _router_body(h_ref, wt_ref, exp_ref, pm_ref, logit_ref, carry_ref):
    c = pl.program_id(1)

    @pl.when(c == 0)
    def _():
        carry_ref[...] = jnp.zeros_like(carry_ref)

    h = h_ref[0]
    wt = wt_ref[...]
    logits = jnp.dot(h, wt, preferred_element_type=jnp.float32)
    logit_ref[0] = logits

    m = jnp.max(logits, axis=-1, keepdims=True)
    s = jnp.sum(jnp.exp(logits - m), axis=-1, keepdims=True)
    pm_ref[0] = 1.0 / s

    eq = (logits >= m).astype(jnp.bfloat16)
    rr = lax.broadcasted_iota(jnp.int32, (_NUM_EXPERTS, _NUM_EXPERTS), 0)
    cc = lax.broadcasted_iota(jnp.int32, (_NUM_EXPERTS, _NUM_EXPERTS), 1)
    upper = (rr <= cc).astype(jnp.bfloat16)
    pfx = jnp.dot(eq, upper, preferred_element_type=jnp.float32)
    ohf = (eq * (pfx <= 1.0).astype(jnp.bfloat16)).astype(jnp.float32)

    pri = _cumsum_rows(ohf) + carry_ref[...]

    keep = (pri <= _CAPACITY).astype(jnp.float32)
    exp_ref[0] = (ohf * keep).astype(jnp.int32)
    carry_ref[...] = carry_ref[...] + jnp.sum(ohf, axis=0, keepdims=True)


def kernel(hidden_states, W):
    B, S, H = hidden_states.shape
    wt = W.T
    grid = (B, S // _CHUNK)
    out_shape = (
        jax.ShapeDtypeStruct((B, S, _NUM_EXPERTS), jnp.int32),
        jax.ShapeDtypeStruct((B, S, 1), jnp.float32),
        jax.ShapeDtypeStruct((B, S, _NUM_EXPERTS), jnp.float32),
    )
    return pl.pallas_call(
        _router_body,
        grid=grid,
        in_specs=[
            pl.BlockSpec((1, _CHUNK, H), lambda b, c: (b, c, 0)),
            pl.BlockSpec((H, _NUM_EXPERTS), lambda b, c: (0, 0)),
        ],
        out_specs=(
            pl.BlockSpec((1, _CHUNK, _NUM_EXPERTS), lambda b, c: (b, c, 0)),
            pl.BlockSpec((1, _CHUNK, 1), lambda b, c: (b, c, 0)),
            pl.BlockSpec((1, _CHUNK, _NUM_EXPERTS), lambda b, c: (b, c, 0)),
        ),
        out_shape=out_shape,
        scratch_shapes=[pltpu.VMEM((1, _NUM_EXPERTS), jnp.float32)],
        compiler_params=pltpu.CompilerParams(
            dimension_semantics=("arbitrary", "arbitrary")),
    )(hidden_states, wt)


# speedup vs baseline: 1.5336x; 1.0017x over previous
import jax
import jax.numpy as jnp
from jax import lax
from jax.experimental import pallas as pl
from jax.experimental.pallas import tpu as pltpu

_NUM_EXPERTS = 16
_CAPACITY = 128.0
_CHUNK = 2048


def _cumsum_rows(x):
    k = 1
    while k < _CHUNK:
        x = x + jnp.pad(x[:-k], ((k, 0), (0, 0)))
        k *= 2
    return x


def _router_body(h_ref, wt_ref, exp_ref, pm_ref, logit_ref, carry_ref):
    c = pl.program_id(1)

    @pl.when(c == 0)
    def _():
        carry_ref[...] = jnp.zeros_like(carry_ref)

    h = h_ref[0]
    wt = wt_ref[...]
    logits = jnp.dot(h, wt, preferred_element_type=jnp.float32)
    logit_ref[0] = logits

    m = jnp.max(logits, axis=-1, keepdims=True)
    s = jnp.sum(jnp.exp(logits - m), axis=-1, keepdims=True)
    pm_ref[0] = 1.0 / s

    eq = (logits >= m).astype(jnp.bfloat16)
    rr = lax.broadcasted_iota(jnp.int32, (_NUM_EXPERTS, _NUM_EXPERTS), 0)
    cc = lax.broadcasted_iota(jnp.int32, (_NUM_EXPERTS, _NUM_EXPERTS), 1)
    upper = (rr <= cc).astype(jnp.bfloat16)
    pfx = jnp.dot(eq, upper, preferred_element_type=jnp.float32)
    ohf = (eq * (pfx <= 1.0).astype(jnp.bfloat16)).astype(jnp.float32)

    pri = _cumsum_rows(ohf) + carry_ref[...]

    keep = (pri <= _CAPACITY).astype(jnp.float32)
    exp_ref[0] = (ohf * keep).astype(jnp.int32)
    carry_ref[...] = carry_ref[...] + jnp.sum(ohf, axis=0, keepdims=True)


def kernel(hidden_states, W):
    B, S, H = hidden_states.shape
    wt = W.T
    grid = (B, S // _CHUNK)
    out_shape = (
        jax.ShapeDtypeStruct((B, S, _NUM_EXPERTS), jnp.int32),
        jax.ShapeDtypeStruct((B, S, 1), jnp.float32),
        jax.ShapeDtypeStruct((B, S, _NUM_EXPERTS), jnp.float32),
    )
    return pl.pallas_call(
        _router_body,
        grid=grid,
        in_specs=[
            pl.BlockSpec((1, _CHUNK, H), lambda b, c: (b, c, 0)),
            pl.BlockSpec((H, _NUM_EXPERTS), lambda b, c: (0, 0)),
        ],
        out_specs=(
            pl.BlockSpec((1, _CHUNK, _NUM_EXPERTS), lambda b, c: (b, c, 0)),
            pl.BlockSpec((1, _CHUNK, 1), lambda b, c: (b, c, 0)),
            pl.BlockSpec((1, _CHUNK, _NUM_EXPERTS), lambda b, c: (b, c, 0)),
        ),
        out_shape=out_shape,
        scratch_shapes=[pltpu.VMEM((1, _NUM_EXPERTS), jnp.float32)],
        compiler_params=pltpu.CompilerParams(
            dimension_semantics=("arbitrary", "arbitrary")),
    )(hidden_states, wt)
